# baseline (speedup 1.0000x reference)
import jax
import jax.numpy as jnp
from jax import lax
from jax.experimental import pallas as pl
from jax.experimental.pallas import tpu as pltpu

_B, _T, _N, _F = 512, 72, 5, 10
_GH = 64
_G3 = 3 * _GH
_NF = _N * _F
_NH = _N * _GH
_EPAD = 32
_TS = 8
_NB = 2
_BB = _B // _NB

_f32 = jnp.float32
_DN = (((1,), (1,)), ((), ()))


def _adj_body(ep_ref, W1_ref, W2_ref, W1c_ref, W2c_ref):
    e = ep_ref[...]
    src = e[0:1, :]
    dst = e[1:2, :]
    iota_ne = lax.broadcasted_iota(jnp.int32, (_N, _EPAD), 0)
    Od = (jnp.broadcast_to(dst, (_N, _EPAD)) == iota_ne).astype(_f32)
    Os = (jnp.broadcast_to(src, (_N, _EPAD)) == iota_ne).astype(_f32)
    Acount = lax.dot_general(Od, Os, _DN, preferred_element_type=_f32)
    eye = (lax.broadcasted_iota(jnp.int32, (_N, _N), 0)
           == lax.broadcasted_iota(jnp.int32, (_N, _N), 1)).astype(_f32)
    ones5 = jnp.ones((_N, _N), _f32)
    degc = jnp.dot(Acount, ones5, preferred_element_type=_f32) + 1.0
    degr = lax.dot_general(eye, degc, _DN, preferred_element_type=_f32)
    A = (Acount + eye) * lax.rsqrt(degc) * lax.rsqrt(degr)

    def onehot(shape, rowfun):
        r = lax.broadcasted_iota(jnp.int32, shape, 0)
        c = lax.broadcasted_iota(jnp.int32, shape, 1)
        return (rowfun(r) == c).astype(_f32)

    E1 = onehot((_NF, _N), lambda r: r // _F)
    F1 = onehot((_NF, _F), lambda r: r % _F)
    E2 = onehot((_NH, _N), lambda r: r // _GH)
    F2 = onehot((_NH, _GH), lambda r: r % _GH)

    P = lax.dot_general(E1, A, _DN, preferred_element_type=_f32)
    P = lax.dot_general(P, E2, _DN, preferred_element_type=_f32)
    Q = jnp.dot(F1, W1_ref[...], preferred_element_type=_f32)
    Q = lax.dot_general(Q, F2, _DN, preferred_element_type=_f32)
    W1c_ref[...] = P * Q

    R = lax.dot_general(E2, A, _DN, preferred_element_type=_f32)
    R = lax.dot_general(R, E1, _DN, preferred_element_type=_f32)
    S = jnp.dot(F2, W2_ref[...], preferred_element_type=_f32)
    S = lax.dot_general(S, F1, _DN, preferred_element_type=_f32)
    W2c_ref[...] = R * S


def _gru_cell(h, gi, Whh_ref, bhh_ref):
    gh = jnp.dot(h, Whh_ref[...], preferred_element_type=_f32) + bhh_ref[...]
    rz = jax.nn.sigmoid(gi[:, 0:2 * _GH] + gh[:, 0:2 * _GH])
    r = rz[:, 0:_GH]
    z = rz[:, _GH:2 * _GH]
    ng = jnp.tanh(gi[:, 2 * _GH:] + r * gh[:, 2 * _GH:])
    return (1.0 - z) * ng + z * h


def _enc_body(x_ref, W1c_ref, b1t_ref, Wih_ref, bih_ref, Whh_ref, bhh_ref,
              lat_ref, h_ref):
    t = pl.program_id(1)

    @pl.when(t == 0)
    def _():
        h_ref[...] = jnp.zeros_like(h_ref)

    h = h_ref[...]
    for k in range(_TS):
        xt = x_ref[k]
        enc = jnp.tanh(jnp.dot(xt, W1c_ref[...], preferred_element_type=_f32)
                       + b1t_ref[...])
        gi = jnp.dot(enc, Wih_ref[...], preferred_element_type=_f32) + bih_ref[...]
        h = _gru_cell(h, gi, Whh_ref, bhh_ref)
    h_ref[...] = h

    @pl.when(t == _T // _TS - 1)
    def _():
        lat_ref[...] = h


def _dec_body(lat_ref, Wir_ref, Wiz_ref, Win_ref, bir_ref, biz_ref, bin_ref,
              Whr_ref, Whz_ref, Whn_ref, bhr_ref, bhz_ref, bhn_ref,
              Wfc_ref, bfc_ref, W2c_ref, b2t_ref, x_ref, out_ref,
              h_ref, gir_ref, giz_ref, gin_ref):
    t = pl.program_id(1)

    @pl.when(t == 0)
    def _():
        lat = lat_ref[...]
        gir_ref[...] = jnp.dot(lat, Wir_ref[...], preferred_element_type=_f32) + bir_ref[...]
        giz_ref[...] = jnp.dot(lat, Wiz_ref[...], preferred_element_type=_f32) + biz_ref[...]
        gin_ref[...] = jnp.dot(lat, Win_ref[...], preferred_element_type=_f32) + bin_ref[...]
        h_ref[...] = jnp.zeros_like(h_ref)

    h = h_ref[...]
    gir, giz, gin = gir_ref[...], giz_ref[...], gin_ref[...]
    for k in range(_TS):
        r = jax.nn.sigmoid(gir + jnp.dot(h, Whr_ref[...], preferred_element_type=_f32) + bhr_ref[...])
        z = jax.nn.sigmoid(giz + jnp.dot(h, Whz_ref[...], preferred_element_type=_f32) + bhz_ref[...])
        ng = jnp.tanh(gin + r * (jnp.dot(h, Whn_ref[...], preferred_element_type=_f32) + bhn_ref[...]))
        h = (1.0 - z) * ng + z * h
        fc = jnp.tanh(jnp.dot(h, Wfc_ref[...], preferred_element_type=_f32)
                      + bfc_ref[...])
        rec = jnp.dot(fc, W2c_ref[...], preferred_element_type=_f32) + b2t_ref[...]
        out_ref[k] = rec + x_ref[k]
    h_ref[...] = h


def _full(shape):
    nd = len(shape)
    return pl.BlockSpec(shape, lambda b, t, _nd=nd: (0,) * _nd)


def kernel(x, edge_index, W1, b1, Wih1, Whh1, bih1, bhh1,
           Wih2, Whh2, bih2, bhh2, Wfc, bfc, W2, b2):
    E = edge_index.shape[1]
    ep = jnp.full((8, _EPAD), 100, jnp.int32)
    ep = ep.at[0, :E].set(edge_index[0])
    ep = ep.at[1, :E].set(edge_index[1])

    W1c, W2c = pl.pallas_call(
        _adj_body,
        out_shape=(jax.ShapeDtypeStruct((_NF, _NH), _f32),
                   jax.ShapeDtypeStruct((_NH, _NF), _f32)),
    )(ep, W1, W2)

    Wih1T, Whh1T = Wih1.T, Whh1.T
    bih1r, bhh1r = bih1[None], bhh1[None]

    def split_w(W):
        return W[0:_GH].T, W[_GH:2 * _GH].T, W[2 * _GH:].T

    def split_b(bb):
        return bb[0:_GH][None], bb[_GH:2 * _GH][None], bb[2 * _GH:][None]

    Wir2, Wiz2, Win2 = split_w(Wih2)
    Whr2, Whz2, Whn2 = split_w(Whh2)
    bir2, biz2, bin2 = split_b(bih2)
    bhr2, bhz2, bhn2 = split_b(bhh2)
    b1t = jnp.tile(b1, _N)[None]
    b2t = jnp.tile(b2, _N)[None]
    bfc2 = bfc[None]

    xT = jnp.swapaxes(x, 0, 1).reshape(_T, _B, _NF)

    xspec = pl.BlockSpec((_TS, _BB, _NF), lambda b, t: (t, b, 0))
    b192 = _full((1, _G3))

    lat = pl.pallas_call(
        _enc_body,
        grid=(_NB, _T // _TS),
        in_specs=[xspec, _full((_NF, _NH)), _full((1, _NH)),
                  _full((_NH, _G3)), b192, _full((_GH, _G3)), b192],
        out_specs=pl.BlockSpec((_BB, _GH), lambda b, t: (b, 0)),
        out_shape=jax.ShapeDtypeStruct((_B, _GH), _f32),
        scratch_shapes=[pltpu.VMEM((_BB, _GH), _f32)],
        compiler_params=pltpu.CompilerParams(
            dimension_semantics=("parallel", "arbitrary")),
    )(xT, W1c, b1t, Wih1T, bih1r, Whh1T, bhh1r)

    g64 = _full((_GH, _GH))
    b64 = _full((1, _GH))
    outT = pl.pallas_call(
        _dec_body,
        grid=(_NB, _T // _TS),
        in_specs=[pl.BlockSpec((_BB, _GH), lambda b, t: (b, 0)),
                  g64, g64, g64, b64, b64, b64,
                  g64, g64, g64, b64, b64, b64,
                  _full((_GH, _NH)), _full((1, _NH)),
                  _full((_NH, _NF)), _full((1, _NF)), xspec],
        out_specs=pl.BlockSpec((_TS, _BB, _NF), lambda b, t: (t, b, 0)),
        out_shape=jax.ShapeDtypeStruct((_T, _B, _NF), _f32),
        scratch_shapes=[pltpu.VMEM((_BB, _GH), _f32)] * 4,
        compiler_params=pltpu.CompilerParams(
            dimension_semantics=("parallel", "arbitrary")),
    )(lat, Wir2, Wiz2, Win2, bir2, biz2, bin2,
      Whr2, Whz2, Whn2, bhr2, bhz2, bhn2, Wfc, bfc2, W2c, b2t, xT)

    return jnp.swapaxes(outT.reshape(_T, _B, _N, _F), 0, 1)


# speedup vs baseline: 1.2170x; 1.2170x over previous
import jax
import jax.numpy as jnp
from jax import lax
from jax.experimental import pallas as pl
from jax.experimental.pallas import tpu as pltpu

_B, _T, _N, _F = 512, 72, 5, 10
_GH = 64
_G3 = 3 * _GH
_NF = _N * _F
_NH = _N * _GH
_EPAD = 32
_TS = 8

_f32 = jnp.float32
_DN = (((1,), (1,)), ((), ()))


def _adj_body(ep_ref, W1_ref, W2_ref, W1c_ref, W2c_ref):
    e = ep_ref[...]
    src = e[0:1, :]
    dst = e[1:2, :]
    iota_ne = lax.broadcasted_iota(jnp.int32, (_N, _EPAD), 0)
    Od = (jnp.broadcast_to(dst, (_N, _EPAD)) == iota_ne).astype(_f32)
    Os = (jnp.broadcast_to(src, (_N, _EPAD)) == iota_ne).astype(_f32)
    Acount = lax.dot_general(Od, Os, _DN, preferred_element_type=_f32)
    eye = (lax.broadcasted_iota(jnp.int32, (_N, _N), 0)
           == lax.broadcasted_iota(jnp.int32, (_N, _N), 1)).astype(_f32)
    ones5 = jnp.ones((_N, _N), _f32)
    degc = jnp.dot(Acount, ones5, preferred_element_type=_f32) + 1.0
    degr = lax.dot_general(eye, degc, _DN, preferred_element_type=_f32)
    A = (Acount + eye) * lax.rsqrt(degc) * lax.rsqrt(degr)

    def onehot(shape, rowfun):
        r = lax.broadcasted_iota(jnp.int32, shape, 0)
        c = lax.broadcasted_iota(jnp.int32, shape, 1)
        return (rowfun(r) == c).astype(_f32)

    E1 = onehot((_NF, _N), lambda r: r // _F)
    F1 = onehot((_NF, _F), lambda r: r % _F)
    E2 = onehot((_NH, _N), lambda r: r // _GH)
    F2 = onehot((_NH, _GH), lambda r: r % _GH)

    P = lax.dot_general(E1, A, _DN, preferred_element_type=_f32)
    P = lax.dot_general(P, E2, _DN, preferred_element_type=_f32)
    Q = jnp.dot(F1, W1_ref[...], preferred_element_type=_f32)
    Q = lax.dot_general(Q, F2, _DN, preferred_element_type=_f32)
    W1c_ref[...] = P * Q

    R = lax.dot_general(E2, A, _DN, preferred_element_type=_f32)
    R = lax.dot_general(R, E1, _DN, preferred_element_type=_f32)
    S = jnp.dot(F2, W2_ref[...], preferred_element_type=_f32)
    S = lax.dot_general(S, F1, _DN, preferred_element_type=_f32)
    W2c_ref[...] = R * S


def _gru_cell(h, gi, Whh_ref, bhh_ref):
    gh = jnp.dot(h, Whh_ref[...], preferred_element_type=_f32) + bhh_ref[...]
    rz = jax.nn.sigmoid(gi[:, 0:2 * _GH] + gh[:, 0:2 * _GH])
    r = rz[:, 0:_GH]
    z = rz[:, _GH:2 * _GH]
    ng = jnp.tanh(gi[:, 2 * _GH:] + r * gh[:, 2 * _GH:])
    return (1.0 - z) * ng + z * h


def _enc_body(x_ref, W1c_ref, b1t_ref, Wih_ref, bih_ref, Whh_ref, bhh_ref,
              lat_ref, h_ref):
    t = pl.program_id(0)

    @pl.when(t == 0)
    def _():
        h_ref[...] = jnp.zeros_like(h_ref)

    h = h_ref[...]
    for k in range(_TS):
        xt = x_ref[k]
        enc = jnp.tanh(jnp.dot(xt, W1c_ref[...], preferred_element_type=_f32)
                       + b1t_ref[...])
        gi = jnp.dot(enc, Wih_ref[...], preferred_element_type=_f32) + bih_ref[...]
        h = _gru_cell(h, gi, Whh_ref, bhh_ref)
    h_ref[...] = h

    @pl.when(t == _T // _TS - 1)
    def _():
        lat_ref[...] = h


def _dec_body(lat_ref, Wir_ref, Wiz_ref, Win_ref, bir_ref, biz_ref, bin_ref,
              Whr_ref, Whz_ref, Whn_ref, bhr_ref, bhz_ref, bhn_ref,
              Wfc_ref, bfc_ref, W2c_ref, b2t_ref, x_ref, out_ref,
              h_ref, gir_ref, giz_ref, gin_ref):
    t = pl.program_id(0)

    @pl.when(t == 0)
    def _():
        lat = lat_ref[...]
        gir_ref[...] = jnp.dot(lat, Wir_ref[...], preferred_element_type=_f32) + bir_ref[...]
        giz_ref[...] = jnp.dot(lat, Wiz_ref[...], preferred_element_type=_f32) + biz_ref[...]
        gin_ref[...] = jnp.dot(lat, Win_ref[...], preferred_element_type=_f32) + bin_ref[...]
        h_ref[...] = jnp.zeros_like(h_ref)

    h = h_ref[...]
    gir, giz, gin = gir_ref[...], giz_ref[...], gin_ref[...]
    for k in range(_TS):
        r = jax.nn.sigmoid(gir + jnp.dot(h, Whr_ref[...], preferred_element_type=_f32) + bhr_ref[...])
        z = jax.nn.sigmoid(giz + jnp.dot(h, Whz_ref[...], preferred_element_type=_f32) + bhz_ref[...])
        ng = jnp.tanh(gin + r * (jnp.dot(h, Whn_ref[...], preferred_element_type=_f32) + bhn_ref[...]))
        h = (1.0 - z) * ng + z * h
        fc = jnp.tanh(jnp.dot(h, Wfc_ref[...], preferred_element_type=_f32)
                      + bfc_ref[...])
        rec = jnp.dot(fc, W2c_ref[...], preferred_element_type=_f32) + b2t_ref[...]
        out_ref[k] = rec + x_ref[k]
    h_ref[...] = h


def _full(shape):
    nd = len(shape)
    return pl.BlockSpec(shape, lambda t, _nd=nd: (0,) * _nd)


def kernel(x, edge_index, W1, b1, Wih1, Whh1, bih1, bhh1,
           Wih2, Whh2, bih2, bhh2, Wfc, bfc, W2, b2):
    E = edge_index.shape[1]
    ep = jnp.full((8, _EPAD), 100, jnp.int32)
    ep = ep.at[0, :E].set(edge_index[0])
    ep = ep.at[1, :E].set(edge_index[1])

    W1c, W2c = pl.pallas_call(
        _adj_body,
        out_shape=(jax.ShapeDtypeStruct((_NF, _NH), _f32),
                   jax.ShapeDtypeStruct((_NH, _NF), _f32)),
    )(ep, W1, W2)

    Wih1T, Whh1T = Wih1.T, Whh1.T
    bih1r, bhh1r = bih1[None], bhh1[None]

    def split_w(W):
        return W[0:_GH].T, W[_GH:2 * _GH].T, W[2 * _GH:].T

    def split_b(bb):
        return bb[0:_GH][None], bb[_GH:2 * _GH][None], bb[2 * _GH:][None]

    Wir2, Wiz2, Win2 = split_w(Wih2)
    Whr2, Whz2, Whn2 = split_w(Whh2)
    bir2, biz2, bin2 = split_b(bih2)
    bhr2, bhz2, bhn2 = split_b(bhh2)
    b1t = jnp.tile(b1, _N)[None]
    b2t = jnp.tile(b2, _N)[None]
    bfc2 = bfc[None]

    xT = jnp.swapaxes(x, 0, 1).reshape(_T, _B, _NF)

    xspec = pl.BlockSpec((_TS, _B, _NF), lambda t: (t, 0, 0))
    b192 = _full((1, _G3))

    xTb = xT.astype(jnp.bfloat16)
    lat = pl.pallas_call(
        _enc_body,
        grid=(_T // _TS,),
        in_specs=[xspec, _full((_NF, _NH)), _full((1, _NH)),
                  _full((_NH, _G3)), b192, _full((_GH, _G3)), b192],
        out_specs=pl.BlockSpec((_B, _GH), lambda t: (0, 0)),
        out_shape=jax.ShapeDtypeStruct((_B, _GH), _f32),
        scratch_shapes=[pltpu.VMEM((_B, _GH), _f32)],
    )(xTb, W1c.astype(jnp.bfloat16), b1t, Wih1T, bih1r, Whh1T, bhh1r)

    g64 = _full((_GH, _GH))
    b64 = _full((1, _GH))
    outT = pl.pallas_call(
        _dec_body,
        grid=(_T // _TS,),
        in_specs=[_full((_B, _GH)), g64, g64, g64, b64, b64, b64,
                  g64, g64, g64, b64, b64, b64,
                  _full((_GH, _NH)), _full((1, _NH)),
                  _full((_NH, _NF)), _full((1, _NF)), xspec],
        out_specs=pl.BlockSpec((_TS, _B, _NF), lambda t: (t, 0, 0)),
        out_shape=jax.ShapeDtypeStruct((_T, _B, _NF), _f32),
        scratch_shapes=[pltpu.VMEM((_B, _GH), _f32)] * 4,
    )(lat, Wir2, Wiz2, Win2, bir2, biz2, bin2,
      Whr2, Whz2, Whn2, bhr2, bhz2, bhn2, Wfc, bfc2, W2c, b2t, xT)

    return jnp.swapaxes(outT.reshape(_T, _B, _N, _F), 0, 1)


# speedup vs baseline: 1.3903x; 1.1424x over previous
import jax
import jax.numpy as jnp
from jax import lax
from jax.experimental import pallas as pl
from jax.experimental.pallas import tpu as pltpu

_B, _T, _N, _F = 512, 72, 5, 10
_GH = 64
_G3 = 3 * _GH
_NF = _N * _F
_NH = _N * _GH
_EPAD = 32
_TS = 8
_HT = _T // _TS

_f32 = jnp.float32
_DN = (((1,), (1,)), ((), ()))


def _build_kron_weights(ep_ref, W1_ref, W2_ref, W1c_ref, W2c_ref):
    e = ep_ref[...]
    src = e[0:1, :]
    dst = e[1:2, :]
    iota_ne = lax.broadcasted_iota(jnp.int32, (_N, _EPAD), 0)
    Od = (jnp.broadcast_to(dst, (_N, _EPAD)) == iota_ne).astype(_f32)
    Os = (jnp.broadcast_to(src, (_N, _EPAD)) == iota_ne).astype(_f32)
    Acount = lax.dot_general(Od, Os, _DN, preferred_element_type=_f32)
    eye = (lax.broadcasted_iota(jnp.int32, (_N, _N), 0)
           == lax.broadcasted_iota(jnp.int32, (_N, _N), 1)).astype(_f32)
    ones5 = jnp.ones((_N, _N), _f32)
    degc = jnp.dot(Acount, ones5, preferred_element_type=_f32) + 1.0
    degr = lax.dot_general(eye, degc, _DN, preferred_element_type=_f32)
    A = (Acount + eye) * lax.rsqrt(degc) * lax.rsqrt(degr)

    def onehot(shape, rowfun):
        r = lax.broadcasted_iota(jnp.int32, shape, 0)
        c = lax.broadcasted_iota(jnp.int32, shape, 1)
        return (rowfun(r) == c).astype(_f32)

    E1 = onehot((_NF, _N), lambda r: r // _F)
    F1 = onehot((_NF, _F), lambda r: r % _F)
    E2 = onehot((_NH, _N), lambda r: r // _GH)
    F2 = onehot((_NH, _GH), lambda r: r % _GH)

    P = lax.dot_general(E1, A, _DN, preferred_element_type=_f32)
    P = lax.dot_general(P, E2, _DN, preferred_element_type=_f32)
    Q = jnp.dot(F1, W1_ref[...], preferred_element_type=_f32)
    Q = lax.dot_general(Q, F2, _DN, preferred_element_type=_f32)
    W1c_ref[...] = P * Q

    R = lax.dot_general(E2, A, _DN, preferred_element_type=_f32)
    R = lax.dot_general(R, E1, _DN, preferred_element_type=_f32)
    S = jnp.dot(F2, W2_ref[...], preferred_element_type=_f32)
    S = lax.dot_general(S, F1, _DN, preferred_element_type=_f32)
    W2c_ref[...] = R * S


def _gru_cell(h, gi, Whh_ref, bhh_ref):
    gh = jnp.dot(h, Whh_ref[...], preferred_element_type=_f32) + bhh_ref[...]
    rz = jax.nn.sigmoid(gi[:, 0:2 * _GH] + gh[:, 0:2 * _GH])
    r = rz[:, 0:_GH]
    z = rz[:, _GH:2 * _GH]
    ng = jnp.tanh(gi[:, 2 * _GH:] + r * gh[:, 2 * _GH:])
    return (1.0 - z) * ng + z * h


def _mega_body(ep_ref, W1_ref, W2_ref, x_ref, b1t_ref, Wih_ref, bih_ref,
               Whh_ref, bhh_ref,
               Wir_ref, Wiz_ref, Win_ref, bir_ref, biz_ref, bin_ref,
               Whr_ref, Whz_ref, Whn_ref, bhr_ref, bhz_ref, bhn_ref,
               Wfc_ref, bfc_ref, b2t_ref, out_ref,
               W1c_s, W2c_s, h_s, gir_s, giz_s, gin_s):
    t = pl.program_id(0)

    @pl.when(t == 0)
    def _():
        _build_kron_weights(ep_ref, W1_ref, W2_ref, W1c_s, W2c_s)
        h_s[...] = jnp.zeros_like(h_s)

    @pl.when(t < _HT)
    def _():
        h = h_s[...]
        for k in range(_TS):
            enc = jnp.tanh(jnp.dot(x_ref[k], W1c_s[...], preferred_element_type=_f32)
                           + b1t_ref[...])
            gi = jnp.dot(enc, Wih_ref[...], preferred_element_type=_f32) + bih_ref[...]
            h = _gru_cell(h, gi, Whh_ref, bhh_ref)
        h_s[...] = h

    @pl.when(t == _HT)
    def _():
        lat = h_s[...]
        gir_s[...] = jnp.dot(lat, Wir_ref[...], preferred_element_type=_f32) + bir_ref[...]
        giz_s[...] = jnp.dot(lat, Wiz_ref[...], preferred_element_type=_f32) + biz_ref[...]
        gin_s[...] = jnp.dot(lat, Win_ref[...], preferred_element_type=_f32) + bin_ref[...]
        h_s[...] = jnp.zeros_like(h_s)

    @pl.when(t >= _HT)
    def _():
        h = h_s[...]
        gir, giz, gin = gir_s[...], giz_s[...], gin_s[...]
        for k in range(_TS):
            r = jax.nn.sigmoid(gir + jnp.dot(h, Whr_ref[...], preferred_element_type=_f32)
                               + bhr_ref[...])
            z = jax.nn.sigmoid(giz + jnp.dot(h, Whz_ref[...], preferred_element_type=_f32)
                               + bhz_ref[...])
            ng = jnp.tanh(gin + r * (jnp.dot(h, Whn_ref[...], preferred_element_type=_f32)
                                     + bhn_ref[...]))
            h = (1.0 - z) * ng + z * h
            fc = jnp.tanh(jnp.dot(h, Wfc_ref[...], preferred_element_type=_f32)
                          + bfc_ref[...])
            rec = jnp.dot(fc, W2c_s[...], preferred_element_type=_f32) + b2t_ref[...]
            out_ref[k] = rec + x_ref[k]
        h_s[...] = h


def _full(shape):
    nd = len(shape)
    return pl.BlockSpec(shape, lambda t, _nd=nd: (0,) * _nd)


def kernel(x, edge_index, W1, b1, Wih1, Whh1, bih1, bhh1,
           Wih2, Whh2, bih2, bhh2, Wfc, bfc, W2, b2):
    E = edge_index.shape[1]
    ep = jnp.full((8, _EPAD), 100, jnp.int32)
    ep = ep.at[0, :E].set(edge_index[0])
    ep = ep.at[1, :E].set(edge_index[1])

    Wih1T, Whh1T = Wih1.T, Whh1.T
    bih1r, bhh1r = bih1[None], bhh1[None]

    def split_w(W):
        return W[0:_GH].T, W[_GH:2 * _GH].T, W[2 * _GH:].T

    def split_b(bb):
        return bb[0:_GH][None], bb[_GH:2 * _GH][None], bb[2 * _GH:][None]

    Wir2, Wiz2, Win2 = split_w(Wih2)
    Whr2, Whz2, Whn2 = split_w(Whh2)
    bir2, biz2, bin2 = split_b(bih2)
    bhr2, bhz2, bhn2 = split_b(bhh2)
    b1t = jnp.tile(b1, _N)[None]
    b2t = jnp.tile(b2, _N)[None]
    bfc2 = bfc[None]

    xT = jnp.swapaxes(x, 0, 1).reshape(_T, _B, _NF)

    xspec = pl.BlockSpec(
        (_TS, _B, _NF),
        lambda t: (jnp.where(t < _HT, t, t - _HT), 0, 0))
    b192 = _full((1, _G3))
    g64 = _full((_GH, _GH))
    b64 = _full((1, _GH))

    outT = pl.pallas_call(
        _mega_body,
        grid=(2 * _HT,),
        in_specs=[_full((8, _EPAD)), _full((_F, _GH)), _full((_GH, _F)),
                  xspec, _full((1, _NH)),
                  _full((_NH, _G3)), b192, _full((_GH, _G3)), b192,
                  g64, g64, g64, b64, b64, b64,
                  g64, g64, g64, b64, b64, b64,
                  _full((_GH, _NH)), _full((1, _NH)), _full((1, _NF))],
        out_specs=pl.BlockSpec(
            (_TS, _B, _NF),
            lambda t: (jnp.where(t < _HT, 0, t - _HT), 0, 0)),
        out_shape=jax.ShapeDtypeStruct((_T, _B, _NF), _f32),
        scratch_shapes=[pltpu.VMEM((_NF, _NH), _f32),
                        pltpu.VMEM((_NH, _NF), _f32),
                        pltpu.VMEM((_B, _GH), _f32),
                        pltpu.VMEM((_B, _GH), _f32),
                        pltpu.VMEM((_B, _GH), _f32),
                        pltpu.VMEM((_B, _GH), _f32)],
    )(ep, W1, W2, xT, b1t, Wih1T, bih1r, Whh1T, bhh1r,
      Wir2, Wiz2, Win2, bir2, biz2, bin2,
      Whr2, Whz2, Whn2, bhr2, bhz2, bhn2,
      Wfc, bfc2, b2t)

    return jnp.swapaxes(outT.reshape(_T, _B, _N, _F), 0, 1)


# speedup vs baseline: 1.4133x; 1.0166x over previous
import jax
import jax.numpy as jnp
from jax import lax
from jax.experimental import pallas as pl
from jax.experimental.pallas import tpu as pltpu

_B, _T, _N, _F = 512, 72, 5, 10
_GH = 64
_G3 = 3 * _GH
_NF = _N * _F
_NH = _N * _GH
_EPAD = 32
_TS = 12
_HT = _T // _TS

_f32 = jnp.float32
_DN = (((1,), (1,)), ((), ()))


def _build_kron_weights(ep_ref, W1_ref, W2_ref, W1c_ref, W2c_ref):
    e = ep_ref[...]
    src = e[0:1, :]
    dst = e[1:2, :]
    iota_ne = lax.broadcasted_iota(jnp.int32, (_N, _EPAD), 0)
    Od = (jnp.broadcast_to(dst, (_N, _EPAD)) == iota_ne).astype(_f32)
    Os = (jnp.broadcast_to(src, (_N, _EPAD)) == iota_ne).astype(_f32)
    Acount = lax.dot_general(Od, Os, _DN, preferred_element_type=_f32)
    eye = (lax.broadcasted_iota(jnp.int32, (_N, _N), 0)
           == lax.broadcasted_iota(jnp.int32, (_N, _N), 1)).astype(_f32)
    ones5 = jnp.ones((_N, _N), _f32)
    degc = jnp.dot(Acount, ones5, preferred_element_type=_f32) + 1.0
    degr = lax.dot_general(eye, degc, _DN, preferred_element_type=_f32)
    A = (Acount + eye) * lax.rsqrt(degc) * lax.rsqrt(degr)

    def onehot(shape, rowfun):
        r = lax.broadcasted_iota(jnp.int32, shape, 0)
        c = lax.broadcasted_iota(jnp.int32, shape, 1)
        return (rowfun(r) == c).astype(_f32)

    E1 = onehot((_NF, _N), lambda r: r // _F)
    F1 = onehot((_NF, _F), lambda r: r % _F)
    E2 = onehot((_NH, _N), lambda r: r // _GH)
    F2 = onehot((_NH, _GH), lambda r: r % _GH)

    P = lax.dot_general(E1, A, _DN, preferred_element_type=_f32)
    P = lax.dot_general(P, E2, _DN, preferred_element_type=_f32)
    Q = jnp.dot(F1, W1_ref[...], preferred_element_type=_f32)
    Q = lax.dot_general(Q, F2, _DN, preferred_element_type=_f32)
    W1c_ref[...] = P * Q

    R = lax.dot_general(E2, A, _DN, preferred_element_type=_f32)
    R = lax.dot_general(R, E1, _DN, preferred_element_type=_f32)
    S = jnp.dot(F2, W2_ref[...], preferred_element_type=_f32)
    S = lax.dot_general(S, F1, _DN, preferred_element_type=_f32)
    W2c_ref[...] = R * S


def _gru_cell(h, gi, Whh_ref, bhh_ref):
    gh = jnp.dot(h, Whh_ref[...], preferred_element_type=_f32) + bhh_ref[...]
    rz = jax.nn.sigmoid(gi[:, 0:2 * _GH] + gh[:, 0:2 * _GH])
    r = rz[:, 0:_GH]
    z = rz[:, _GH:2 * _GH]
    ng = jnp.tanh(gi[:, 2 * _GH:] + r * gh[:, 2 * _GH:])
    return (1.0 - z) * ng + z * h


def _mega_body(ep_ref, W1_ref, W2_ref, x_ref, b1t_ref, Wih_ref, bih_ref,
               Whh_ref, bhh_ref,
               Wir_ref, Wiz_ref, Win_ref, bir_ref, biz_ref, bin_ref,
               Whr_ref, Whz_ref, Whn_ref, bhr_ref, bhz_ref, bhn_ref,
               Wfc_ref, bfc_ref, b2t_ref, out_ref,
               W1c_s, W2c_s, h_s, gir_s, giz_s, gin_s):
    t = pl.program_id(0)

    @pl.when(t == 0)
    def _():
        _build_kron_weights(ep_ref, W1_ref, W2_ref, W1c_s, W2c_s)
        h_s[...] = jnp.zeros_like(h_s)

    @pl.when(t < _HT)
    def _():
        h = h_s[...]
        for k in range(_TS):
            enc = jnp.tanh(jnp.dot(x_ref[k], W1c_s[...], preferred_element_type=_f32)
                           + b1t_ref[...])
            gi = jnp.dot(enc, Wih_ref[...], preferred_element_type=_f32) + bih_ref[...]
            h = _gru_cell(h, gi, Whh_ref, bhh_ref)
        h_s[...] = h

    @pl.when(t == _HT)
    def _():
        lat = h_s[...]
        gir_s[...] = jnp.dot(lat, Wir_ref[...], preferred_element_type=_f32) + bir_ref[...]
        giz_s[...] = jnp.dot(lat, Wiz_ref[...], preferred_element_type=_f32) + biz_ref[...]
        gin_s[...] = jnp.dot(lat, Win_ref[...], preferred_element_type=_f32) + bin_ref[...]
        h_s[...] = jnp.zeros_like(h_s)

    @pl.when(t >= _HT)
    def _():
        h = h_s[...]
        gir, giz, gin = gir_s[...], giz_s[...], gin_s[...]
        for k in range(_TS):
            r = jax.nn.sigmoid(gir + jnp.dot(h, Whr_ref[...], preferred_element_type=_f32)
                               + bhr_ref[...])
            z = jax.nn.sigmoid(giz + jnp.dot(h, Whz_ref[...], preferred_element_type=_f32)
                               + bhz_ref[...])
            ng = jnp.tanh(gin + r * (jnp.dot(h, Whn_ref[...], preferred_element_type=_f32)
                                     + bhn_ref[...]))
            h = (1.0 - z) * ng + z * h
            fc = jnp.tanh(jnp.dot(h, Wfc_ref[...], preferred_element_type=_f32)
                          + bfc_ref[...])
            rec = jnp.dot(fc, W2c_s[...], preferred_element_type=_f32) + b2t_ref[...]
            out_ref[k] = rec + x_ref[k]
        h_s[...] = h


def _full(shape):
    nd = len(shape)
    return pl.BlockSpec(shape, lambda t, _nd=nd: (0,) * _nd)


def kernel(x, edge_index, W1, b1, Wih1, Whh1, bih1, bhh1,
           Wih2, Whh2, bih2, bhh2, Wfc, bfc, W2, b2):
    E = edge_index.shape[1]
    ep = jnp.full((8, _EPAD), 100, jnp.int32)
    ep = ep.at[0, :E].set(edge_index[0])
    ep = ep.at[1, :E].set(edge_index[1])

    Wih1T, Whh1T = Wih1.T, Whh1.T
    bih1r, bhh1r = bih1[None], bhh1[None]

    def split_w(W):
        return W[0:_GH].T, W[_GH:2 * _GH].T, W[2 * _GH:].T

    def split_b(bb):
        return bb[0:_GH][None], bb[_GH:2 * _GH][None], bb[2 * _GH:][None]

    Wir2, Wiz2, Win2 = split_w(Wih2)
    Whr2, Whz2, Whn2 = split_w(Whh2)
    bir2, biz2, bin2 = split_b(bih2)
    bhr2, bhz2, bhn2 = split_b(bhh2)
    b1t = jnp.tile(b1, _N)[None]
    b2t = jnp.tile(b2, _N)[None]
    bfc2 = bfc[None]

    xT = jnp.swapaxes(x, 0, 1).reshape(_T, _B, _NF)

    xspec = pl.BlockSpec(
        (_TS, _B, _NF),
        lambda t: (jnp.where(t < _HT, t, t - _HT), 0, 0))
    b192 = _full((1, _G3))
    g64 = _full((_GH, _GH))
    b64 = _full((1, _GH))

    outT = pl.pallas_call(
        _mega_body,
        grid=(2 * _HT,),
        in_specs=[_full((8, _EPAD)), _full((_F, _GH)), _full((_GH, _F)),
                  xspec, _full((1, _NH)),
                  _full((_NH, _G3)), b192, _full((_GH, _G3)), b192,
                  g64, g64, g64, b64, b64, b64,
                  g64, g64, g64, b64, b64, b64,
                  _full((_GH, _NH)), _full((1, _NH)), _full((1, _NF))],
        out_specs=pl.BlockSpec(
            (_TS, _B, _NF),
            lambda t: (jnp.where(t < _HT, 0, t - _HT), 0, 0)),
        out_shape=jax.ShapeDtypeStruct((_T, _B, _NF), _f32),
        scratch_shapes=[pltpu.VMEM((_NF, _NH), _f32),
                        pltpu.VMEM((_NH, _NF), _f32),
                        pltpu.VMEM((_B, _GH), _f32),
                        pltpu.VMEM((_B, _GH), _f32),
                        pltpu.VMEM((_B, _GH), _f32),
                        pltpu.VMEM((_B, _GH), _f32)],
    )(ep, W1, W2, xT, b1t, Wih1T, bih1r, Whh1T, bhh1r,
      Wir2, Wiz2, Win2, bir2, biz2, bin2,
      Whr2, Whz2, Whn2, bhr2, bhz2, bhn2,
      Wfc, bfc2, b2t)

    return jnp.swapaxes(outT.reshape(_T, _B, _N, _F), 0, 1)


# speedup vs baseline: 1.4196x; 1.0045x over previous
import jax
import jax.numpy as jnp
from jax import lax
from jax.experimental import pallas as pl
from jax.experimental.pallas import tpu as pltpu

_B, _T, _N, _F = 512, 72, 5, 10
_GH = 64
_G3 = 3 * _GH
_NF = _N * _F
_NH = _N * _GH
_EPAD = 32
_TS = 18
_HT = _T // _TS

_f32 = jnp.float32
_DN = (((1,), (1,)), ((), ()))


def _build_kron_weights(ep_ref, W1_ref, W2_ref, W1c_ref, W2c_ref):
    e = ep_ref[...]
    src = e[0:1, :]
    dst = e[1:2, :]
    iota_ne = lax.broadcasted_iota(jnp.int32, (_N, _EPAD), 0)
    Od = (jnp.broadcast_to(dst, (_N, _EPAD)) == iota_ne).astype(_f32)
    Os = (jnp.broadcast_to(src, (_N, _EPAD)) == iota_ne).astype(_f32)
    Acount = lax.dot_general(Od, Os, _DN, preferred_element_type=_f32)
    eye = (lax.broadcasted_iota(jnp.int32, (_N, _N), 0)
           == lax.broadcasted_iota(jnp.int32, (_N, _N), 1)).astype(_f32)
    ones5 = jnp.ones((_N, _N), _f32)
    degc = jnp.dot(Acount, ones5, preferred_element_type=_f32) + 1.0
    degr = lax.dot_general(eye, degc, _DN, preferred_element_type=_f32)
    A = (Acount + eye) * lax.rsqrt(degc) * lax.rsqrt(degr)

    def onehot(shape, rowfun):
        r = lax.broadcasted_iota(jnp.int32, shape, 0)
        c = lax.broadcasted_iota(jnp.int32, shape, 1)
        return (rowfun(r) == c).astype(_f32)

    E1 = onehot((_NF, _N), lambda r: r // _F)
    F1 = onehot((_NF, _F), lambda r: r % _F)
    E2 = onehot((_NH, _N), lambda r: r // _GH)
    F2 = onehot((_NH, _GH), lambda r: r % _GH)

    P = lax.dot_general(E1, A, _DN, preferred_element_type=_f32)
    P = lax.dot_general(P, E2, _DN, preferred_element_type=_f32)
    Q = jnp.dot(F1, W1_ref[...], preferred_element_type=_f32)
    Q = lax.dot_general(Q, F2, _DN, preferred_element_type=_f32)
    W1c_ref[...] = P * Q

    R = lax.dot_general(E2, A, _DN, preferred_element_type=_f32)
    R = lax.dot_general(R, E1, _DN, preferred_element_type=_f32)
    S = jnp.dot(F2, W2_ref[...], preferred_element_type=_f32)
    S = lax.dot_general(S, F1, _DN, preferred_element_type=_f32)
    W2c_ref[...] = R * S


def _gru_cell(h, gi, Whh_ref, bhh_ref):
    gh = jnp.dot(h, Whh_ref[...], preferred_element_type=_f32) + bhh_ref[...]
    rz = jax.nn.sigmoid(gi[:, 0:2 * _GH] + gh[:, 0:2 * _GH])
    r = rz[:, 0:_GH]
    z = rz[:, _GH:2 * _GH]
    ng = jnp.tanh(gi[:, 2 * _GH:] + r * gh[:, 2 * _GH:])
    return (1.0 - z) * ng + z * h


def _mega_body(ep_ref, W1_ref, W2_ref, x_ref, b1t_ref, Wih_ref, bih_ref,
               Whh_ref, bhh_ref,
               Wir_ref, Wiz_ref, Win_ref, bir_ref, biz_ref, bin_ref,
               Whr_ref, Whz_ref, Whn_ref, bhr_ref, bhz_ref, bhn_ref,
               Wfc_ref, bfc_ref, b2t_ref, out_ref,
               W1c_s, W2c_s, h_s, gir_s, giz_s, gin_s):
    t = pl.program_id(0)

    @pl.when(t == 0)
    def _():
        _build_kron_weights(ep_ref, W1_ref, W2_ref, W1c_s, W2c_s)
        h_s[...] = jnp.zeros_like(h_s)

    @pl.when(t < _HT)
    def _():
        h = h_s[...]
        for k in range(_TS):
            enc = jnp.tanh(jnp.dot(x_ref[k], W1c_s[...], preferred_element_type=_f32)
                           + b1t_ref[...])
            gi = jnp.dot(enc, Wih_ref[...], preferred_element_type=_f32) + bih_ref[...]
            h = _gru_cell(h, gi, Whh_ref, bhh_ref)
        h_s[...] = h

    @pl.when(t == _HT)
    def _():
        lat = h_s[...]
        gir_s[...] = jnp.dot(lat, Wir_ref[...], preferred_element_type=_f32) + bir_ref[...]
        giz_s[...] = jnp.dot(lat, Wiz_ref[...], preferred_element_type=_f32) + biz_ref[...]
        gin_s[...] = jnp.dot(lat, Win_ref[...], preferred_element_type=_f32) + bin_ref[...]
        h_s[...] = jnp.zeros_like(h_s)

    @pl.when(t >= _HT)
    def _():
        h = h_s[...]
        gir, giz, gin = gir_s[...], giz_s[...], gin_s[...]
        for k in range(_TS):
            r = jax.nn.sigmoid(gir + jnp.dot(h, Whr_ref[...], preferred_element_type=_f32)
                               + bhr_ref[...])
            z = jax.nn.sigmoid(giz + jnp.dot(h, Whz_ref[...], preferred_element_type=_f32)
                               + bhz_ref[...])
            ng = jnp.tanh(gin + r * (jnp.dot(h, Whn_ref[...], preferred_element_type=_f32)
                                     + bhn_ref[...]))
            h = (1.0 - z) * ng + z * h
            fc = jnp.tanh(jnp.dot(h, Wfc_ref[...], preferred_element_type=_f32)
                          + bfc_ref[...])
            rec = jnp.dot(fc, W2c_s[...], preferred_element_type=_f32) + b2t_ref[...]
            out_ref[k] = rec + x_ref[k]
        h_s[...] = h


def _full(shape):
    nd = len(shape)
    return pl.BlockSpec(shape, lambda t, _nd=nd: (0,) * _nd)


def kernel(x, edge_index, W1, b1, Wih1, Whh1, bih1, bhh1,
           Wih2, Whh2, bih2, bhh2, Wfc, bfc, W2, b2):
    E = edge_index.shape[1]
    ep = jnp.full((8, _EPAD), 100, jnp.int32)
    ep = ep.at[0, :E].set(edge_index[0])
    ep = ep.at[1, :E].set(edge_index[1])

    Wih1T, Whh1T = Wih1.T, Whh1.T
    bih1r, bhh1r = bih1[None], bhh1[None]

    def split_w(W):
        return W[0:_GH].T, W[_GH:2 * _GH].T, W[2 * _GH:].T

    def split_b(bb):
        return bb[0:_GH][None], bb[_GH:2 * _GH][None], bb[2 * _GH:][None]

    Wir2, Wiz2, Win2 = split_w(Wih2)
    Whr2, Whz2, Whn2 = split_w(Whh2)
    bir2, biz2, bin2 = split_b(bih2)
    bhr2, bhz2, bhn2 = split_b(bhh2)
    b1t = jnp.tile(b1, _N)[None]
    b2t = jnp.tile(b2, _N)[None]
    bfc2 = bfc[None]

    xT = jnp.swapaxes(x, 0, 1).reshape(_T, _B, _NF)

    xspec = pl.BlockSpec(
        (_TS, _B, _NF),
        lambda t: (jnp.where(t < _HT, t, t - _HT), 0, 0))
    b192 = _full((1, _G3))
    g64 = _full((_GH, _GH))
    b64 = _full((1, _GH))

    outT = pl.pallas_call(
        _mega_body,
        grid=(2 * _HT,),
        in_specs=[_full((8, _EPAD)), _full((_F, _GH)), _full((_GH, _F)),
                  xspec, _full((1, _NH)),
                  _full((_NH, _G3)), b192, _full((_GH, _G3)), b192,
                  g64, g64, g64, b64, b64, b64,
                  g64, g64, g64, b64, b64, b64,
                  _full((_GH, _NH)), _full((1, _NH)), _full((1, _NF))],
        out_specs=pl.BlockSpec(
            (_TS, _B, _NF),
            lambda t: (jnp.where(t < _HT, 0, t - _HT), 0, 0)),
        out_shape=jax.ShapeDtypeStruct((_T, _B, _NF), _f32),
        scratch_shapes=[pltpu.VMEM((_NF, _NH), _f32),
                        pltpu.VMEM((_NH, _NF), _f32),
                        pltpu.VMEM((_B, _GH), _f32),
                        pltpu.VMEM((_B, _GH), _f32),
                        pltpu.VMEM((_B, _GH), _f32),
                        pltpu.VMEM((_B, _GH), _f32)],
    )(ep, W1, W2, xT, b1t, Wih1T, bih1r, Whh1T, bhh1r,
      Wir2, Wiz2, Win2, bir2, biz2, bin2,
      Whr2, Whz2, Whn2, bhr2, bhz2, bhn2,
      Wfc, bfc2, b2t)

    return jnp.swapaxes(outT.reshape(_T, _B, _N, _F), 0, 1)
